# baseline (speedup 1.0000x reference)
import jax
import jax.numpy as jnp
from jax.experimental import pallas as pl
from jax.experimental.pallas import tpu as pltpu

_BM = 400


def _feat_kernel(x_ref, w_ref, h_ref):
    h_ref[:] = jnp.dot(
        x_ref[:].astype(jnp.bfloat16), w_ref[:],
        preferred_element_type=jnp.float32).astype(jnp.bfloat16)


def _layer1_kernel(a_ref, h_ref, w2_ref, g_ref):
    ah = jnp.dot(a_ref[:].astype(jnp.bfloat16), h_ref[:],
                 preferred_element_type=jnp.float32)
    g = jnp.maximum(ah, 0.0).astype(jnp.bfloat16)
    g_ref[:] = jnp.dot(g, w2_ref[:],
                       preferred_element_type=jnp.float32).astype(jnp.bfloat16)


def _layer2_kernel(a_ref, g_ref, o_ref):
    o_ref[:] = jnp.dot(a_ref[:].astype(jnp.bfloat16), g_ref[:],
                       preferred_element_type=jnp.float32)


def kernel(x, adj_low, adj_high, W1, W2):
    n, _ = x.shape
    nhid = W1.shape[1]
    ncls = W2.shape[1]
    w1b = W1.astype(jnp.bfloat16)
    w2b = W2.astype(jnp.bfloat16)

    h = pl.pallas_call(
        _feat_kernel,
        out_shape=jax.ShapeDtypeStruct((n, nhid), jnp.bfloat16),
    )(x, w1b)

    grid = (n // _BM,)
    g = pl.pallas_call(
        _layer1_kernel,
        grid=grid,
        in_specs=[
            pl.BlockSpec((_BM, n), lambda i: (i, 0)),
            pl.BlockSpec((n, nhid), lambda i: (0, 0)),
            pl.BlockSpec((nhid, ncls), lambda i: (0, 0)),
        ],
        out_specs=pl.BlockSpec((_BM, ncls), lambda i: (i, 0)),
        out_shape=jax.ShapeDtypeStruct((n, ncls), jnp.bfloat16),
        compiler_params=pltpu.CompilerParams(
            dimension_semantics=("parallel",)),
    )(adj_low, h, w2b)

    out = pl.pallas_call(
        _layer2_kernel,
        grid=grid,
        in_specs=[
            pl.BlockSpec((_BM, n), lambda i: (i, 0)),
            pl.BlockSpec((n, ncls), lambda i: (0, 0)),
        ],
        out_specs=pl.BlockSpec((_BM, ncls), lambda i: (i, 0)),
        out_shape=jax.ShapeDtypeStruct((n, ncls), jnp.float32),
        compiler_params=pltpu.CompilerParams(
            dimension_semantics=("parallel",)),
    )(adj_low, g)
    return out
